# baseline (device time: 20094 ns/iter reference)
import jax
import jax.numpy as jnp
from jax import lax
from jax.experimental import pallas as pl
from jax.experimental.pallas import tpu as pltpu

N_DEV = 16
N_ROUNDS = 4
N_CHUNK = 4
MASKS = [[1, 3, 4, 8], [4, 8, 1, 3]]


def kernel(t, W):
    m, k = t.shape
    _, n = W.shape
    mh = m // 2
    mq = mh // N_CHUNK

    def body(t_hbm, w_hbm, out_ref, t_vmem, w_vmem, in_sems, out_vmem,
             out_sems, send_ref, recv_ref, send_sems, recv_sems):
        my = lax.axis_index("i")

        barrier_sem = pltpu.get_barrier_semaphore()
        for mask in MASKS[0]:
            pl.semaphore_signal(
                barrier_sem, inc=1,
                device_id=(my ^ mask,), device_id_type=pl.DeviceIdType.MESH,
            )

        t_cp = pltpu.make_async_copy(t_hbm, t_vmem, in_sems.at[0])
        w_cp = pltpu.make_async_copy(w_hbm, w_vmem, in_sems.at[1])
        t_cp.start()
        w_cp.start()
        t_cp.wait()
        w_cp.wait()

        partial = jnp.dot(
            t_vmem[...].astype(jnp.bfloat16),
            w_vmem[...].astype(jnp.bfloat16),
            preferred_element_type=jnp.float32,
        ).astype(jnp.bfloat16)

        pl.semaphore_wait(barrier_sem, N_ROUNDS)

        def make_rdma(h, r, c):
            return pltpu.make_async_remote_copy(
                src_ref=send_ref.at[h, r, c],
                dst_ref=recv_ref.at[h, r, c],
                send_sem=send_sems.at[h, r, c],
                recv_sem=recv_sems.at[h, r, c],
                device_id=(my ^ MASKS[h][r],),
                device_id_type=pl.DeviceIdType.MESH,
            )

        rdmas = {}
        for c in range(N_CHUNK):
            for h in range(2):
                row0 = h * mh + c * mq
                send_ref[h, 0, c] = partial[row0:row0 + mq]
                rdmas[h, 0, c] = make_rdma(h, 0, c)
                rdmas[h, 0, c].start()
        for r in range(1, N_ROUNDS):
            for c in range(N_CHUNK):
                for h in range(2):
                    rdmas[h, r - 1, c].wait_recv()
                    send_ref[h, r, c] = (
                        send_ref[h, r - 1, c] + recv_ref[h, r - 1, c]
                    )
                    rdmas[h, r, c] = make_rdma(h, r, c)
                    rdmas[h, r, c].start()
        last = N_ROUNDS - 1
        out_cps = []
        for c in range(N_CHUNK):
            for h in range(2):
                rdmas[h, last, c].wait_recv()
                row0 = h * mh + c * mq
                out_vmem[row0:row0 + mq] = (
                    send_ref[h, last, c] + recv_ref[h, last, c]
                )
                cp = pltpu.make_async_copy(
                    out_vmem.at[row0:row0 + mq],
                    out_ref.at[row0:row0 + mq],
                    out_sems.at[h, c],
                )
                cp.start()
                out_cps.append(cp)

        for cp in out_cps:
            cp.wait()
        for rdma in rdmas.values():
            rdma.wait_send()

    return pl.pallas_call(
        body,
        out_shape=jax.ShapeDtypeStruct((m, n), jnp.bfloat16),
        in_specs=[
            pl.BlockSpec(memory_space=pl.ANY),
            pl.BlockSpec(memory_space=pl.ANY),
        ],
        out_specs=pl.BlockSpec(memory_space=pl.ANY),
        scratch_shapes=[
            pltpu.VMEM((m, k), t.dtype),
            pltpu.VMEM((k, n), W.dtype),
            pltpu.SemaphoreType.DMA((2,)),
            pltpu.VMEM((m, n), jnp.bfloat16),
            pltpu.SemaphoreType.DMA((2, N_CHUNK)),
            pltpu.VMEM((2, N_ROUNDS, N_CHUNK, mq, n), jnp.bfloat16),
            pltpu.VMEM((2, N_ROUNDS, N_CHUNK, mq, n), jnp.bfloat16),
            pltpu.SemaphoreType.DMA((2, N_ROUNDS, N_CHUNK)),
            pltpu.SemaphoreType.DMA((2, N_ROUNDS, N_CHUNK)),
        ],
        compiler_params=pltpu.CompilerParams(collective_id=0),
    )(t, W)


# device time: 19926 ns/iter; 1.0084x vs baseline; 1.0084x over previous
import jax
import jax.numpy as jnp
from jax import lax
from jax.experimental import pallas as pl
from jax.experimental.pallas import tpu as pltpu

N_DEV = 16
N_ROUNDS = 4
N_CHUNK = 4
MASKS = [[1, 3, 4, 8], [4, 8, 1, 3]]


def kernel(t, W):
    m, k = t.shape
    _, n = W.shape
    mh = m // 2
    mq = mh // N_CHUNK

    def body(t_hbm, w_hbm, out_ref, t_vmem, w_vmem, in_sems,
             send_ref, recv_ref, send_sems, recv_sems):
        my = lax.axis_index("i")

        barrier_sem = pltpu.get_barrier_semaphore()
        for mask in MASKS[0]:
            pl.semaphore_signal(
                barrier_sem, inc=1,
                device_id=(my ^ mask,), device_id_type=pl.DeviceIdType.MESH,
            )

        t_cp = pltpu.make_async_copy(t_hbm, t_vmem, in_sems.at[0])
        w_cp = pltpu.make_async_copy(w_hbm, w_vmem, in_sems.at[1])
        t_cp.start()
        w_cp.start()
        t_cp.wait()
        w_cp.wait()

        partial = jnp.dot(
            t_vmem[...].astype(jnp.bfloat16),
            w_vmem[...].astype(jnp.bfloat16),
            preferred_element_type=jnp.float32,
        ).astype(jnp.bfloat16)

        def make_rdma(h, r, c):
            return pltpu.make_async_remote_copy(
                src_ref=send_ref.at[h, r, c],
                dst_ref=recv_ref.at[h, r, c],
                send_sem=send_sems.at[h, r, c],
                recv_sem=recv_sems.at[h, r, c],
                device_id=(my ^ MASKS[h][r],),
                device_id_type=pl.DeviceIdType.MESH,
            )

        rdmas = {}
        for c in range(N_CHUNK):
            for h in range(2):
                row0 = h * mh + c * mq
                send_ref[h, 0, c] = partial[row0:row0 + mq]
                rdmas[h, 0, c] = make_rdma(h, 0, c)

        pl.semaphore_wait(barrier_sem, N_ROUNDS)

        for c in range(N_CHUNK):
            for h in range(2):
                rdmas[h, 0, c].start()
        for r in range(1, N_ROUNDS):
            for c in range(N_CHUNK):
                for h in range(2):
                    rdmas[h, r - 1, c].wait_recv()
                    send_ref[h, r, c] = (
                        send_ref[h, r - 1, c] + recv_ref[h, r - 1, c]
                    )
                    rdmas[h, r, c] = make_rdma(h, r, c)
                    rdmas[h, r, c].start()
        last = N_ROUNDS - 1
        for c in range(N_CHUNK):
            for h in range(2):
                rdmas[h, last, c].wait_recv()
                row0 = h * mh + c * mq
                out_ref[row0:row0 + mq] = (
                    send_ref[h, last, c] + recv_ref[h, last, c]
                )

        for rdma in rdmas.values():
            rdma.wait_send()

    return pl.pallas_call(
        body,
        out_shape=jax.ShapeDtypeStruct((m, n), jnp.bfloat16),
        in_specs=[
            pl.BlockSpec(memory_space=pl.ANY),
            pl.BlockSpec(memory_space=pl.ANY),
        ],
        out_specs=pl.BlockSpec(memory_space=pltpu.VMEM),
        scratch_shapes=[
            pltpu.VMEM((m, k), t.dtype),
            pltpu.VMEM((k, n), W.dtype),
            pltpu.SemaphoreType.DMA((2,)),
            pltpu.VMEM((2, N_ROUNDS, N_CHUNK, mq, n), jnp.bfloat16),
            pltpu.VMEM((2, N_ROUNDS, N_CHUNK, mq, n), jnp.bfloat16),
            pltpu.SemaphoreType.DMA((2, N_ROUNDS, N_CHUNK)),
            pltpu.SemaphoreType.DMA((2, N_ROUNDS, N_CHUNK)),
        ],
        compiler_params=pltpu.CompilerParams(collective_id=0),
    )(t, W)
